# Initial kernel scaffold; baseline (speedup 1.0000x reference)
import jax
import jax.numpy as jnp
from jax.experimental import pallas as pl


def kernel(xyz, grid, xyz_min, xyz_max):
    raise NotImplementedError("write your pallas kernel here")


# speedup vs baseline: 3.6487x; 3.6487x over previous
import jax
import jax.numpy as jnp
from jax import lax
from jax.experimental import pallas as pl
from jax.experimental.pallas import tpu as pltpu
from jax.experimental.pallas import tpu_sc as plsc

N_PTS = 1048576
C = 16
D = H = W = 128
DHW = D * H * W
NC, NS, L = 2, 16, 16
NW = NC * NS
PER_W = N_PTS // NW
B = 128
NCHUNK = PER_W // B


def _sc_body(pts_hbm, table_hbm, out_hbm, pts_v, idx_v, fr_v, rows_v, acc_v, sem):
    wid = lax.axis_index("s") * NC + lax.axis_index("c")
    base = wid * PER_W

    def chunk(j, carry):
        off = base + j * B
        pltpu.sync_copy(pts_hbm.at[:, pl.ds(off, B)], pts_v)

        def grp(g, c2):
            s = pl.ds(g * L, L)
            x = pts_v[0, s]
            y = pts_v[1, s]
            z = pts_v[2, s]
            xi = jnp.minimum(jnp.maximum(x.astype(jnp.int32), 0), W - 2)
            yi = jnp.minimum(jnp.maximum(y.astype(jnp.int32), 0), H - 2)
            zi = jnp.minimum(jnp.maximum(z.astype(jnp.int32), 0), D - 2)
            fr_v[pl.ds(g * L, L)] = x - xi.astype(jnp.float32)
            fr_v[pl.ds(B + g * L, L)] = y - yi.astype(jnp.float32)
            fr_v[pl.ds(2 * B + g * L, L)] = z - zi.astype(jnp.float32)
            f000 = zi * (H * W) + yi * W + xi
            idx_v[0, s] = f000
            idx_v[1, s] = f000 + 1
            idx_v[2, s] = f000 + W
            idx_v[3, s] = f000 + (W + 1)
            idx_v[4, s] = f000 + H * W
            idx_v[5, s] = f000 + (H * W + 1)
            idx_v[6, s] = f000 + (H * W + W)
            idx_v[7, s] = f000 + (H * W + W + 1)
            return c2

        lax.fori_loop(0, B // L, grp, 0)

        cps = [
            pltpu.async_copy(table_hbm.at[idx_v.at[t]], rows_v.at[t], sem)
            for t in range(8)
        ]
        for cp in cps:
            cp.wait()

        def pt(b, c2):
            bb = jnp.full((L,), b, jnp.int32)
            fx = plsc.load_gather(fr_v, [bb])
            fy = plsc.load_gather(fr_v, [bb + B])
            fz = plsc.load_gather(fr_v, [bb + 2 * B])
            v000 = rows_v[0, b, :]
            v001 = rows_v[1, b, :]
            v010 = rows_v[2, b, :]
            v011 = rows_v[3, b, :]
            v100 = rows_v[4, b, :]
            v101 = rows_v[5, b, :]
            v110 = rows_v[6, b, :]
            v111 = rows_v[7, b, :]
            a00 = v000 + fx * (v001 - v000)
            a01 = v010 + fx * (v011 - v010)
            a10 = v100 + fx * (v101 - v100)
            a11 = v110 + fx * (v111 - v110)
            b0 = a00 + fy * (a01 - a00)
            b1 = a10 + fy * (a11 - a10)
            acc_v[b, :] = b0 + fz * (b1 - b0)
            return c2

        lax.fori_loop(0, B, pt, 0)

        pltpu.sync_copy(acc_v, out_hbm.at[pl.ds(off, B)])
        return carry

    lax.fori_loop(0, NCHUNK, chunk, 0)


_sc_call = pl.kernel(
    _sc_body,
    out_type=jax.ShapeDtypeStruct((N_PTS, C), jnp.float32),
    mesh=plsc.VectorSubcoreMesh(
        core_axis_name="c", subcore_axis_name="s", num_cores=NC, num_subcores=NS
    ),
    scratch_types=[
        pltpu.VMEM((3, B), jnp.float32),
        pltpu.VMEM((8, B), jnp.int32),
        pltpu.VMEM((3 * B,), jnp.float32),
        pltpu.VMEM((8, B, C), jnp.float32),
        pltpu.VMEM((B, C), jnp.float32),
        pltpu.SemaphoreType.DMA,
    ],
    compiler_params=pltpu.CompilerParams(
        needs_layout_passes=False, use_tc_tiling_on_sc=False
    ),
)


@jax.jit
def kernel(xyz, grid, xyz_min, xyz_max):
    shape = xyz.shape[:-1]
    pts = xyz.reshape(-1, 3)
    ind = (pts - xyz_min) / (xyz_max - xyz_min) * 2.0 - 1.0
    scale = jnp.array([W - 1, H - 1, D - 1], jnp.float32)
    p = (ind + 1.0) * 0.5 * scale
    pts_t = p.T
    table = grid[0].reshape(C, DHW).T
    out = _sc_call(pts_t, table)
    return out.reshape(*shape, C)
